# baseline (speedup 1.0000x reference)
import functools

import jax
import jax.numpy as jnp
from jax import lax
from jax.experimental import pallas as pl
from jax.experimental.pallas import tpu as pltpu
from jax.experimental.pallas import tpu_sc as plsc

NA = 100000
NE = 6400000
C = 2048
NCHUNKS = NE // C
NC = 2
NS = 16
NW = NC * NS
NT = -(-NCHUNKS // NW)
ZR = 6256
ZR_LAST = NA - (NS - 1) * ZR


def _sc_body(pi_h, pj_h, dx_h, z_h, outp_h, outn_h,
             pib, pjb, dxcb, dxb,
             px, py, pz, nx, ny, nz, sem):
    c = lax.axis_index("c")
    s = lax.axis_index("s")
    w = s * NC + c
    planes_p = (px, py, pz)
    planes_n = (nx, ny, nz)

    def init(off, ln):
        for a in planes_p + planes_n:
            pltpu.sync_copy(z_h.at[pl.ds(off, ln)], a.at[pl.ds(off, ln)])

    @pl.when(s < NS - 1)
    def _():
        init(s * ZR, ZR)

    @pl.when(s == NS - 1)
    def _():
        init((NS - 1) * ZR, ZR_LAST)

    plsc.subcore_barrier()

    def chunk_body(t, carry):
        k = t * NW + w

        @pl.when(k < NCHUNKS)
        def _():
            e0 = k * C
            pltpu.sync_copy(pi_h.at[pl.ds(e0, C)], pib)
            pltpu.sync_copy(pj_h.at[pl.ds(e0, C)], pjb)
            pltpu.sync_copy(dx_h.at[pl.ds(e0, C)], dxcb)

            tbl = lax.iota(jnp.int32, 16)

            def xp(v, carry):
                rows = tbl + 16 * v
                for j in range(3):
                    cols = jnp.broadcast_to(j, (16,)).astype(jnp.int32)
                    dxb[j, pl.ds(16 * v, 16)] = plsc.load_gather(
                        dxcb, [rows, cols])
                return carry

            lax.fori_loop(0, C // 16, xp, 0)
            descs = []
            for j in range(3):
                descs.append(pltpu.async_copy(
                    dxb.at[j], planes_p[j].at[pib], sem, add=True))
                descs.append(pltpu.async_copy(
                    dxb.at[j], planes_n[j].at[pjb], sem, add=True))
            for d in descs:
                d.wait()

        return carry

    lax.fori_loop(0, NT, chunk_body, 0)

    plsc.subcore_barrier()

    def drain(off, ln):
        for j in range(3):
            pltpu.sync_copy(planes_p[j].at[pl.ds(off, ln)],
                            outp_h.at[c, j, pl.ds(off, ln)])
            pltpu.sync_copy(planes_n[j].at[pl.ds(off, ln)],
                            outn_h.at[c, j, pl.ds(off, ln)])

    @pl.when(s < NS - 1)
    def _():
        drain(s * ZR, ZR)

    @pl.when(s == NS - 1)
    def _():
        drain((NS - 1) * ZR, ZR_LAST)


_sc_scatter = functools.partial(
    pl.kernel,
    out_type=[
        jax.ShapeDtypeStruct((NC, 3, NA), jnp.float32),
        jax.ShapeDtypeStruct((NC, 3, NA), jnp.float32),
    ],
    mesh=plsc.VectorSubcoreMesh(core_axis_name="c", subcore_axis_name="s"),
    compiler_params=pltpu.CompilerParams(use_tc_tiling_on_sc=False,
                                         needs_layout_passes=False),
    scratch_types=[
        pltpu.VMEM((C,), jnp.int32),
        pltpu.VMEM((C,), jnp.int32),
        pltpu.VMEM((C, 3), jnp.float32),
        pltpu.VMEM((3, C), jnp.float32),
        pltpu.VMEM_SHARED((NA,), jnp.float32),
        pltpu.VMEM_SHARED((NA,), jnp.float32),
        pltpu.VMEM_SHARED((NA,), jnp.float32),
        pltpu.VMEM_SHARED((NA,), jnp.float32),
        pltpu.VMEM_SHARED((NA,), jnp.float32),
        pltpu.VMEM_SHARED((NA,), jnp.float32),
        pltpu.SemaphoreType.DMA,
    ],
)(_sc_body)


_CB = 8192


def _combine_body(p_ref, n_ref, o_ref):
    o_ref[...] = (p_ref[0:3] + p_ref[3:6]) - (n_ref[0:3] + n_ref[3:6])


_combine = pl.pallas_call(
    _combine_body,
    grid=(-(-NA // _CB),),
    in_specs=[
        pl.BlockSpec((NC * 3, _CB), lambda i: (0, i)),
        pl.BlockSpec((NC * 3, _CB), lambda i: (0, i)),
    ],
    out_specs=pl.BlockSpec((3, _CB), lambda i: (0, i)),
    out_shape=jax.ShapeDtypeStruct((3, NA), jnp.float32),
)


def kernel(dx, pair_i, pair_j):
    zeros = jnp.zeros((NA,), jnp.float32)
    part_p, part_n = _sc_scatter(pair_i, pair_j, dx, zeros)
    planes = _combine(part_p.reshape(NC * 3, NA), part_n.reshape(NC * 3, NA))
    return planes.T


# speedup vs baseline: 1.7823x; 1.7823x over previous
import functools

import jax
import jax.numpy as jnp
from jax import lax
from jax.experimental import pallas as pl
from jax.experimental.pallas import tpu as pltpu
from jax.experimental.pallas import tpu_sc as plsc

NA = 100000
NE = 6400000
C = 2048
NCHUNKS = NE // C
NC = 2
NS = 16
NW = NC * NS
NT = -(-NCHUNKS // NW)
ZR = 6256
ZR_LAST = NA - (NS - 1) * ZR


def _sc_body(pi_h, pj_h, dxt_h, z_h, outp_h, outn_h,
             pib, pjb, dxb,
             px, py, pz, nx, ny, nz, sem):
    c = lax.axis_index("c")
    s = lax.axis_index("s")
    w = s * NC + c
    planes_p = (px, py, pz)
    planes_n = (nx, ny, nz)

    def init(off, ln):
        for a in planes_p + planes_n:
            pltpu.sync_copy(z_h.at[pl.ds(off, ln)], a.at[pl.ds(off, ln)])

    @pl.when(s < NS - 1)
    def _():
        init(s * ZR, ZR)

    @pl.when(s == NS - 1)
    def _():
        init((NS - 1) * ZR, ZR_LAST)

    plsc.subcore_barrier()

    def chunk_body(t, carry):
        k = t * NW + w

        @pl.when(k < NCHUNKS)
        def _():
            e0 = k * C
            pltpu.sync_copy(pi_h.at[pl.ds(e0, C)], pib)
            pltpu.sync_copy(pj_h.at[pl.ds(e0, C)], pjb)
            for j in range(3):
                pltpu.sync_copy(dxt_h.at[j, pl.ds(e0, C)], dxb.at[j])
            descs = []
            for j in range(3):
                descs.append(pltpu.async_copy(
                    dxb.at[j], planes_p[j].at[pib], sem, add=True))
                descs.append(pltpu.async_copy(
                    dxb.at[j], planes_n[j].at[pjb], sem, add=True))
            for d in descs:
                d.wait()

        return carry

    lax.fori_loop(0, NT, chunk_body, 0)

    plsc.subcore_barrier()

    def drain(off, ln):
        for j in range(3):
            pltpu.sync_copy(planes_p[j].at[pl.ds(off, ln)],
                            outp_h.at[c, j, pl.ds(off, ln)])
            pltpu.sync_copy(planes_n[j].at[pl.ds(off, ln)],
                            outn_h.at[c, j, pl.ds(off, ln)])

    @pl.when(s < NS - 1)
    def _():
        drain(s * ZR, ZR)

    @pl.when(s == NS - 1)
    def _():
        drain((NS - 1) * ZR, ZR_LAST)


_sc_scatter = functools.partial(
    pl.kernel,
    out_type=[
        jax.ShapeDtypeStruct((NC, 3, NA), jnp.float32),
        jax.ShapeDtypeStruct((NC, 3, NA), jnp.float32),
    ],
    mesh=plsc.VectorSubcoreMesh(core_axis_name="c", subcore_axis_name="s"),
    compiler_params=pltpu.CompilerParams(use_tc_tiling_on_sc=False),
    scratch_types=[
        pltpu.VMEM((C,), jnp.int32),
        pltpu.VMEM((C,), jnp.int32),
        pltpu.VMEM((3, C), jnp.float32),
        pltpu.VMEM_SHARED((NA,), jnp.float32),
        pltpu.VMEM_SHARED((NA,), jnp.float32),
        pltpu.VMEM_SHARED((NA,), jnp.float32),
        pltpu.VMEM_SHARED((NA,), jnp.float32),
        pltpu.VMEM_SHARED((NA,), jnp.float32),
        pltpu.VMEM_SHARED((NA,), jnp.float32),
        pltpu.SemaphoreType.DMA,
    ],
)(_sc_body)


_XB = 16000


def _xpose_body(x_ref, e_ref, o_ref):
    o_ref[...] = jax.lax.dot_general(
        e_ref[...], x_ref[...], (((1,), (1,)), ((), ())),
        precision=jax.lax.Precision.HIGHEST,
        preferred_element_type=jnp.float32)


_xpose = pl.pallas_call(
    _xpose_body,
    grid=(NE // _XB,),
    in_specs=[
        pl.BlockSpec((_XB, 3), lambda i: (i, 0)),
        pl.BlockSpec((3, 3), lambda i: (0, 0)),
    ],
    out_specs=pl.BlockSpec((3, _XB), lambda i: (0, i)),
    out_shape=jax.ShapeDtypeStruct((3, NE), jnp.float32),
)


_CB = 8192


def _combine_body(p_ref, n_ref, o_ref):
    o_ref[...] = (p_ref[0:3] + p_ref[3:6]) - (n_ref[0:3] + n_ref[3:6])


_combine = pl.pallas_call(
    _combine_body,
    grid=(-(-NA // _CB),),
    in_specs=[
        pl.BlockSpec((NC * 3, _CB), lambda i: (0, i)),
        pl.BlockSpec((NC * 3, _CB), lambda i: (0, i)),
    ],
    out_specs=pl.BlockSpec((3, _CB), lambda i: (0, i)),
    out_shape=jax.ShapeDtypeStruct((3, NA), jnp.float32),
)


def kernel(dx, pair_i, pair_j):
    dxt = _xpose(dx, jnp.eye(3, dtype=jnp.float32))
    zeros = jnp.zeros((NA,), jnp.float32)
    part_p, part_n = _sc_scatter(pair_i, pair_j, dxt, zeros)
    planes = _combine(part_p.reshape(NC * 3, NA), part_n.reshape(NC * 3, NA))
    return planes.T


# speedup vs baseline: 13.2735x; 7.4476x over previous
import functools

import jax
import jax.numpy as jnp
from jax import lax
from jax.experimental import pallas as pl
from jax.experimental.pallas import tpu as pltpu
from jax.experimental.pallas import tpu_sc as plsc

NA = 100000
NE = 6400000
C = 2048
NCHUNKS = NE // C
NC = 2
NS = 16
NW = NC * NS
NT = -(-NCHUNKS // NW)
ZR = 6256
ZR_LAST = NA - (NS - 1) * ZR


def _sc_body(pi_h, pj_h, d0_h, d1_h, d2_h, z_h, outp_h, outn_h,
             pib, pjb, dxb,
             px, py, pz, nx, ny, nz, sem):
    c = lax.axis_index("c")
    s = lax.axis_index("s")
    w = s * NC + c
    planes_p = (px, py, pz)
    planes_n = (nx, ny, nz)

    def init(off, ln):
        for a in planes_p + planes_n:
            pltpu.sync_copy(z_h.at[pl.ds(off, ln)], a.at[pl.ds(off, ln)])

    @pl.when(s < NS - 1)
    def _():
        init(s * ZR, ZR)

    @pl.when(s == NS - 1)
    def _():
        init((NS - 1) * ZR, ZR_LAST)

    plsc.subcore_barrier()

    def chunk_body(t, carry):
        k = t * NW + w

        @pl.when(k < NCHUNKS)
        def _():
            e0 = k * C
            pltpu.sync_copy(pi_h.at[pl.ds(e0, C)], pib)
            pltpu.sync_copy(pj_h.at[pl.ds(e0, C)], pjb)
            for j, dj_h in enumerate((d0_h, d1_h, d2_h)):
                pltpu.sync_copy(dj_h.at[pl.ds(e0, C)], dxb.at[j])
            descs = []
            for j in range(3):
                descs.append(pltpu.async_copy(
                    dxb.at[j], planes_p[j].at[pib], sem, add=True))
                descs.append(pltpu.async_copy(
                    dxb.at[j], planes_n[j].at[pjb], sem, add=True))
            for d in descs:
                d.wait()

        return carry

    lax.fori_loop(0, NT, chunk_body, 0)

    plsc.subcore_barrier()

    def drain(off, ln):
        for j in range(3):
            pltpu.sync_copy(planes_p[j].at[pl.ds(off, ln)],
                            outp_h.at[c, j, pl.ds(off, ln)])
            pltpu.sync_copy(planes_n[j].at[pl.ds(off, ln)],
                            outn_h.at[c, j, pl.ds(off, ln)])

    @pl.when(s < NS - 1)
    def _():
        drain(s * ZR, ZR)

    @pl.when(s == NS - 1)
    def _():
        drain((NS - 1) * ZR, ZR_LAST)


_sc_scatter = functools.partial(
    pl.kernel,
    out_type=[
        jax.ShapeDtypeStruct((NC, 3, NA), jnp.float32),
        jax.ShapeDtypeStruct((NC, 3, NA), jnp.float32),
    ],
    mesh=plsc.VectorSubcoreMesh(core_axis_name="c", subcore_axis_name="s"),
    compiler_params=pltpu.CompilerParams(use_tc_tiling_on_sc=False),
    scratch_types=[
        pltpu.VMEM((C,), jnp.int32),
        pltpu.VMEM((C,), jnp.int32),
        pltpu.VMEM((3, C), jnp.float32),
        pltpu.VMEM_SHARED((NA,), jnp.float32),
        pltpu.VMEM_SHARED((NA,), jnp.float32),
        pltpu.VMEM_SHARED((NA,), jnp.float32),
        pltpu.VMEM_SHARED((NA,), jnp.float32),
        pltpu.VMEM_SHARED((NA,), jnp.float32),
        pltpu.VMEM_SHARED((NA,), jnp.float32),
        pltpu.SemaphoreType.DMA,
    ],
)(_sc_body)


_CB = 8192


def _combine_body(p_ref, n_ref, o_ref):
    o_ref[...] = (p_ref[0:3] + p_ref[3:6]) - (n_ref[0:3] + n_ref[3:6])


_combine = pl.pallas_call(
    _combine_body,
    grid=(-(-NA // _CB),),
    in_specs=[
        pl.BlockSpec((NC * 3, _CB), lambda i: (0, i)),
        pl.BlockSpec((NC * 3, _CB), lambda i: (0, i)),
    ],
    out_specs=pl.BlockSpec((3, _CB), lambda i: (0, i)),
    out_shape=jax.ShapeDtypeStruct((3, NA), jnp.float32),
)


def kernel(dx, pair_i, pair_j):
    d0, d1, d2 = dx[:, 0], dx[:, 1], dx[:, 2]
    zeros = jnp.zeros((NA,), jnp.float32)
    part_p, part_n = _sc_scatter(pair_i, pair_j, d0, d1, d2, zeros)
    planes = _combine(part_p.reshape(NC * 3, NA), part_n.reshape(NC * 3, NA))
    return planes.T
